# baseline (speedup 1.0000x reference)
import functools
import jax
import jax.numpy as jnp
from jax import lax
from jax.experimental import pallas as pl
from jax.experimental.pallas import tpu as pltpu
from jax.experimental.pallas import tpu_sc as plsc

N = 10000
E = 320000
D = 128
NC, NS = 2, 16
NW = NC * NS
EPW = E // NW
NPADD = 10240
RPT = NPADD // NS

CH1 = 128
NF1 = EPW // CH1
T1 = EPW - NF1 * CH1
G1 = 6
NG1 = NF1 // G1

CH3 = 40
SCH3 = 5
NSCH3 = EPW // (CH3 * SCH3)
NB3 = 5
ZCH = 128
ZC3 = RPT // ZCH


def _sc_mesh():
    return plsc.VectorSubcoreMesh(
        core_axis_name="c", subcore_axis_name="s", num_cores=NC, num_subcores=NS
    )


@functools.partial(
    pl.kernel,
    out_type=jax.ShapeDtypeStruct((NC, NPADD), jnp.float32),
    mesh=_sc_mesh(),
    scratch_types=[
        pltpu.VMEM((EPW,), jnp.int32),
        pltpu.VMEM((CH1,), jnp.float32),
        pltpu.VMEM((RPT,), jnp.float32),
        pltpu.VMEM_SHARED((NPADD,), jnp.float32),
        pltpu.SemaphoreType.DMA,
    ],
)
def _deg_kernel(row_hbm, deg_out, idx_all, ones_v, zer_v, acc_sh, sem):
    cid = lax.axis_index("c")
    sid = lax.axis_index("s")
    wid = sid * NC + cid

    @pl.loop(0, RPT // 16)
    def _zinit(k):
        zer_v[pl.ds(k * 16, 16)] = jnp.zeros((16,), jnp.float32)

    @pl.loop(0, CH1 // 16)
    def _oinit(k):
        ones_v[pl.ds(k * 16, 16)] = jnp.full((16,), 1.0, jnp.float32)

    pltpu.sync_copy(zer_v, acc_sh.at[pl.ds(sid * RPT, RPT)])
    pltpu.sync_copy(row_hbm.at[pl.ds(wid * EPW, EPW)], idx_all)
    plsc.subcore_barrier()

    @pl.loop(0, NG1)
    def _scat(g):
        for k in range(G1):
            idx = idx_all.at[pl.ds((g * G1 + k) * CH1, CH1)]
            pltpu.async_copy(ones_v, acc_sh.at[idx], sem, add=True)
        for k in range(G1):
            idx = idx_all.at[pl.ds((g * G1 + k) * CH1, CH1)]
            pltpu.make_async_copy(ones_v, acc_sh.at[idx], sem).wait()

    tidx = idx_all.at[pl.ds(NF1 * CH1, T1)]
    pltpu.sync_copy(ones_v.at[pl.ds(0, T1)], acc_sh.at[tidx], add=True)

    plsc.subcore_barrier()
    pltpu.sync_copy(acc_sh.at[pl.ds(sid * RPT, RPT)],
                    deg_out.at[cid, pl.ds(sid * RPT, RPT)])


def _mm_body(x_ref, w_ref, degt_ref, hp_ref, dis_ref):
    h = lax.dot_general(x_ref[...], w_ref[...], (((1,), (1,)), ((), ())),
                        preferred_element_type=jnp.float32)
    deg = degt_ref[:, 0:1] + degt_ref[:, 1:2]
    dis = jnp.where(deg > 0.0, lax.rsqrt(deg), 0.0)
    hp_ref[...] = h * dis
    dis_ref[...] = dis


_MM_R = 2000


def _mm_call(x, w, degt):
    return pl.pallas_call(
        _mm_body,
        grid=(N // _MM_R,),
        in_specs=[
            pl.BlockSpec((_MM_R, D), lambda i: (i, 0)),
            pl.BlockSpec((D, D), lambda i: (0, 0)),
            pl.BlockSpec((_MM_R, NC), lambda i: (i, 0)),
        ],
        out_specs=[
            pl.BlockSpec((_MM_R, D), lambda i: (i, 0)),
            pl.BlockSpec((_MM_R, 1), lambda i: (i, 0)),
        ],
        out_shape=[
            jax.ShapeDtypeStruct((N, D), jnp.float32),
            jax.ShapeDtypeStruct((N, 1), jnp.float32),
        ],
    )(x, w, degt)


@functools.partial(
    pl.kernel,
    out_type=jax.ShapeDtypeStruct((NC, NPADD, D), jnp.float32),
    mesh=_sc_mesh(),
    scratch_types=[
        pltpu.VMEM((SCH3 * CH3,), jnp.int32),
        pltpu.VMEM((SCH3 * CH3,), jnp.int32),
        pltpu.VMEM((SCH3 * CH3,), jnp.int32),
        pltpu.VMEM((SCH3 * CH3,), jnp.int32),
        pltpu.VMEM((NB3 * CH3, D), jnp.float32),
        pltpu.VMEM_SHARED((NPADD, D), jnp.float32),
        pltpu.SemaphoreType.DMA,
        pltpu.SemaphoreType.DMA,
        pltpu.SemaphoreType.DMA,
        pltpu.SemaphoreType.DMA,
        pltpu.SemaphoreType.DMA,
        pltpu.SemaphoreType.DMA,
        pltpu.SemaphoreType.DMA,
        pltpu.SemaphoreType.DMA,
        pltpu.SemaphoreType.DMA,
        pltpu.SemaphoreType.DMA,
        pltpu.SemaphoreType.DMA,
        pltpu.SemaphoreType.DMA,
    ],
)
def _prop_kernel(hp_hbm, row_hbm, col_hbm, out_hbm, ridx0, ridx1, cidx0,
                 cidx1, rows_v, acc_sh, semg0, semg1, semg2, semg3, semg4,
                 sems0, sems1, sems2, sems3, sems4, semi, semz):
    cid = lax.axis_index("c")
    sid = lax.axis_index("s")
    wid = sid * NC + cid
    semg = (semg0, semg1, semg2, semg3, semg4)
    sems = (sems0, sems1, sems2, sems3, sems4)

    def buf(b):
        return rows_v.at[pl.ds(b * CH3, CH3)]

    def islot(arr, slot):
        return arr.at[pl.ds(slot * CH3, CH3)]

    def gather(r_arr, slot, b):
        pltpu.async_copy(hp_hbm.at[islot(r_arr, slot)], buf(b), semg[b])

    def gather_wait(b):
        pltpu.make_async_copy(hp_hbm.at[islot(ridx0, 0)], buf(b),
                              semg[b]).wait()

    def scatter(c_arr, slot, b):
        pltpu.async_copy(buf(b), acc_sh.at[islot(c_arr, slot)], sems[b],
                         add=True)

    def scatter_wait(b):
        pltpu.make_async_copy(buf(b), acc_sh.at[islot(cidx0, 0)],
                              sems[b]).wait()

    def load_idx(sc, r_arr, c_arr):
        src = pl.ds(wid * EPW + sc * SCH3 * CH3, SCH3 * CH3)
        pltpu.async_copy(row_hbm.at[src], r_arr, semi)
        pltpu.async_copy(col_hbm.at[src], c_arr, semi)

    def load_idx_wait(r_arr, c_arr):
        src = pl.ds(wid * EPW, SCH3 * CH3)
        pltpu.make_async_copy(row_hbm.at[src], r_arr, semi).wait()
        pltpu.make_async_copy(col_hbm.at[src], c_arr, semi).wait()

    @pl.loop(0, ZCH)
    def _zr(r):
        @pl.loop(0, D // 16)
        def _zc(k):
            rows_v[r, pl.ds(k * 16, 16)] = jnp.zeros((16,), jnp.float32)

    for j in range(ZC3):
        pltpu.async_copy(rows_v.at[pl.ds(0, ZCH)],
                         acc_sh.at[pl.ds(sid * RPT + j * ZCH, ZCH)], semz)
    for j in range(ZC3):
        pltpu.make_async_copy(rows_v.at[pl.ds(0, ZCH)],
                              acc_sh.at[pl.ds(0, ZCH)], semz).wait()

    src0 = pl.ds(wid * EPW, SCH3 * CH3)
    pltpu.sync_copy(row_hbm.at[src0], ridx0)
    pltpu.sync_copy(col_hbm.at[src0], cidx0)
    load_idx(1, ridx1, cidx1)
    gather(ridx0, 0, 0)
    gather(ridx0, 1, 1)
    plsc.subcore_barrier()

    HT = NSCH3 // 2

    @pl.loop(0, HT)
    def _super(t):
        for half in range(2):
            if half == 0:
                cur_r, cur_c, nxt_r, nxt_c = ridx0, cidx0, ridx1, cidx1
            else:
                cur_r, cur_c, nxt_r, nxt_c = ridx1, cidx1, ridx0, cidx0
            for k in range(SCH3):
                b = k
                bw = (k + 2) % NB3
                if half == 0 and k < 3:
                    @pl.when(t > 0)
                    def _w():
                        scatter_wait(bw)
                else:
                    scatter_wait(bw)
                if k == 2:
                    if half == 0:
                        @pl.when(t > 0)
                        def _pf0():
                            load_idx(2 * t + 1, nxt_r, nxt_c)
                    else:
                        @pl.when(t < HT - 1)
                        def _pf1():
                            load_idx(2 * t + 2, nxt_r, nxt_c)
                if k <= SCH3 - 3:
                    gather(cur_r, k + 2, bw)
                elif half == 0:
                    if k == SCH3 - 2:
                        load_idx_wait(nxt_r, nxt_c)
                    gather(nxt_r, k - (SCH3 - 2), bw)
                else:
                    if k == SCH3 - 2:
                        @pl.when(t < HT - 1)
                        def _wi():
                            load_idx_wait(nxt_r, nxt_c)
                    @pl.when(t < HT - 1)
                    def _g():
                        gather(nxt_r, k - (SCH3 - 2), bw)
                gather_wait(b)
                scatter(cur_c, k, b)

    scatter_wait(2)
    scatter_wait(3)
    scatter_wait(4)
    plsc.subcore_barrier()

    for j in range(ZC3):
        r0 = sid * RPT + j * ZCH
        pltpu.async_copy(acc_sh.at[pl.ds(r0, ZCH)],
                         out_hbm.at[cid, pl.ds(r0, ZCH)], semz)
    for j in range(ZC3):
        pltpu.make_async_copy(acc_sh.at[pl.ds(0, ZCH)],
                              out_hbm.at[cid, pl.ds(0, ZCH)], semz).wait()


def _out_body(p_ref, dis_ref, b_ref, o_ref):
    s = p_ref[0] + p_ref[1]
    o_ref[...] = s * dis_ref[...] + b_ref[...]


_OUT_R = 2000


def _out_call(parts, dis, bias2d):
    return pl.pallas_call(
        _out_body,
        grid=(N // _OUT_R,),
        in_specs=[
            pl.BlockSpec((NC, _OUT_R, D), lambda i: (0, i, 0)),
            pl.BlockSpec((_OUT_R, 1), lambda i: (i, 0)),
            pl.BlockSpec((1, D), lambda i: (0, 0)),
        ],
        out_specs=pl.BlockSpec((_OUT_R, D), lambda i: (i, 0)),
        out_shape=jax.ShapeDtypeStruct((N, D), jnp.float32),
    )(parts, dis, bias2d)


def kernel(x, edge_index, adj_norm_sp, W, bias):
    row = edge_index[0].astype(jnp.int32)
    col = edge_index[1].astype(jnp.int32)
    deg_part = _deg_kernel(row)
    hp, dis = _mm_call(x, W, jnp.transpose(deg_part))
    parts = _prop_kernel(hp, row, col)
    out = _out_call(parts, dis, bias.reshape(1, D))
    return out


# speedup vs baseline: 1.0442x; 1.0442x over previous
import functools
import jax
import jax.numpy as jnp
from jax import lax
from jax.experimental import pallas as pl
from jax.experimental.pallas import tpu as pltpu
from jax.experimental.pallas import tpu_sc as plsc

N = 10000
E = 320000
D = 128
NC, NS = 2, 16
NW = NC * NS
EPW = 10240
E_PAD = NW * EPW
NPAD = 10240
RPT = NPAD // NS

CH = 128
CPW = EPW // CH
SCH = 8
NSC = CPW // SCH

CH3 = 80
CPW3 = EPW // CH3
NSCH3 = CPW3 // SCH
NB3 = 4
ZC3 = RPT // CH3


def _sc_mesh():
    return plsc.VectorSubcoreMesh(
        core_axis_name="c", subcore_axis_name="s", num_cores=NC, num_subcores=NS
    )


@functools.partial(
    pl.kernel,
    out_type=jax.ShapeDtypeStruct((NC, NPAD), jnp.float32),
    mesh=_sc_mesh(),
    scratch_types=[
        pltpu.VMEM((CPW, CH), jnp.int32),
        pltpu.VMEM((CH,), jnp.float32),
        pltpu.VMEM((RPT,), jnp.float32),
        pltpu.VMEM_SHARED((NPAD,), jnp.float32),
        pltpu.SemaphoreType.DMA,
    ],
)
def _deg_kernel(row_hbm, deg_out, idx_all, ones_v, zer_v, acc_sh, sem):
    cid = lax.axis_index("c")
    sid = lax.axis_index("s")
    wid = sid * NC + cid

    @pl.loop(0, RPT // 16)
    def _zinit(k):
        zer_v[pl.ds(k * 16, 16)] = jnp.zeros((16,), jnp.float32)

    @pl.loop(0, CH // 16)
    def _oinit(k):
        ones_v[pl.ds(k * 16, 16)] = jnp.full((16,), 1.0, jnp.float32)

    pltpu.sync_copy(zer_v, acc_sh.at[pl.ds(sid * RPT, RPT)])
    pltpu.sync_copy(row_hbm.at[wid], idx_all)
    plsc.subcore_barrier()

    @pl.loop(0, NSC)
    def _scat(s):
        for k in range(SCH):
            pltpu.async_copy(ones_v, acc_sh.at[idx_all.at[s * SCH + k]], sem,
                             add=True)
        for k in range(SCH):
            pltpu.make_async_copy(ones_v, acc_sh.at[idx_all.at[s * SCH + k]],
                                  sem).wait()

    plsc.subcore_barrier()
    pltpu.sync_copy(acc_sh.at[pl.ds(sid * RPT, RPT)],
                    deg_out.at[cid, pl.ds(sid * RPT, RPT)])


def _mm_body(x_ref, w_ref, degt_ref, hp_ref, dis_ref):
    h = lax.dot_general(x_ref[...], w_ref[...], (((1,), (1,)), ((), ())),
                        preferred_element_type=jnp.float32)
    deg = degt_ref[:, 0:1] + degt_ref[:, 1:2]
    dis = jnp.where(deg > 0.0, lax.rsqrt(deg), 0.0)
    hp_ref[...] = h * dis
    dis_ref[...] = dis


_MM_R = 2000


def _mm_call(x, w, degt):
    return pl.pallas_call(
        _mm_body,
        grid=(N // _MM_R,),
        in_specs=[
            pl.BlockSpec((_MM_R, D), lambda i: (i, 0)),
            pl.BlockSpec((D, D), lambda i: (0, 0)),
            pl.BlockSpec((_MM_R, 2), lambda i: (i, 0)),
        ],
        out_specs=[
            pl.BlockSpec((_MM_R, D), lambda i: (i, 0)),
            pl.BlockSpec((_MM_R, 1), lambda i: (i, 0)),
        ],
        out_shape=[
            jax.ShapeDtypeStruct((NPAD, D), jnp.float32),
            jax.ShapeDtypeStruct((NPAD, 1), jnp.float32),
        ],
    )(x, w, degt)


@functools.partial(
    pl.kernel,
    out_type=jax.ShapeDtypeStruct((NC, NPAD, D), jnp.float32),
    mesh=_sc_mesh(),
    scratch_types=[
        pltpu.VMEM((2, SCH, CH3), jnp.int32),
        pltpu.VMEM((2, SCH, CH3), jnp.int32),
        pltpu.VMEM((NB3 * CH3, D), jnp.float32),
        pltpu.VMEM_SHARED((NPAD, D), jnp.float32),
        pltpu.SemaphoreType.DMA,
        pltpu.SemaphoreType.DMA,
        pltpu.SemaphoreType.DMA,
        pltpu.SemaphoreType.DMA,
        pltpu.SemaphoreType.DMA,
        pltpu.SemaphoreType.DMA,
        pltpu.SemaphoreType.DMA,
        pltpu.SemaphoreType.DMA,
        pltpu.SemaphoreType.DMA,
        pltpu.SemaphoreType.DMA,
    ],
)
def _prop_kernel(hp_hbm, row_hbm, col_hbm, out_hbm, ridx, cidx, rows_v,
                 acc_sh, semg0, semg1, semg2, semg3, sems0, sems1, sems2,
                 sems3, semi, semz):
    cid = lax.axis_index("c")
    sid = lax.axis_index("s")
    wid = sid * NC + cid
    semg = (semg0, semg1, semg2, semg3)
    sems = (sems0, sems1, sems2, sems3)

    def buf(b):
        return rows_v.at[pl.ds(b * CH3, CH3)]

    def gather(p, slot, b):
        pltpu.async_copy(hp_hbm.at[ridx.at[p, slot]], buf(b), semg[b])

    def gather_wait(b):
        pltpu.make_async_copy(hp_hbm.at[ridx.at[0, 0]], buf(b),
                              semg[b]).wait()

    def scatter(p, slot, b):
        pltpu.async_copy(buf(b), acc_sh.at[cidx.at[p, slot]], sems[b],
                         add=True)

    def scatter_wait(b):
        pltpu.make_async_copy(buf(b), acc_sh.at[cidx.at[0, 0]],
                              sems[b]).wait()

    def load_idx(sc, p):
        pltpu.async_copy(row_hbm.at[wid, pl.ds(sc * SCH, SCH)], ridx.at[p],
                         semi)
        pltpu.async_copy(col_hbm.at[wid, pl.ds(sc * SCH, SCH)], cidx.at[p],
                         semi)

    @pl.loop(0, CH3)
    def _zr(r):
        @pl.loop(0, D // 16)
        def _zc(k):
            rows_v[r, pl.ds(k * 16, 16)] = jnp.zeros((16,), jnp.float32)

    for j in range(ZC3):
        pltpu.async_copy(buf(0), acc_sh.at[pl.ds(sid * RPT + j * CH3, CH3)],
                         semz)
    for j in range(ZC3):
        pltpu.make_async_copy(buf(0), acc_sh.at[pl.ds(0, CH3)], semz).wait()

    pltpu.sync_copy(row_hbm.at[wid, pl.ds(0, SCH)], ridx.at[0])
    pltpu.sync_copy(col_hbm.at[wid, pl.ds(0, SCH)], cidx.at[0])
    load_idx(1, 1)
    gather(0, 0, 0)
    gather(0, 1, 1)
    plsc.subcore_barrier()

    @pl.loop(0, NSCH3)
    def _super(s):
        p = s % 2
        for k in range(SCH):
            b = k % NB3
            bw = (k + 2) % NB3
            if k < 2:
                @pl.when(s > 0)
                def _w():
                    scatter_wait(bw)
            else:
                scatter_wait(bw)
            if k == 2:
                @pl.when((s > 0) & (s < NSCH3 - 1))
                def _pf():
                    load_idx(s + 1, 1 - p)
            if k <= SCH - 3:
                gather(p, k + 2, bw)
            else:
                if k == SCH - 2:
                    @pl.when(s < NSCH3 - 1)
                    def _wi():
                        pltpu.make_async_copy(
                            row_hbm.at[wid, pl.ds(0, SCH)], ridx.at[1 - p],
                            semi).wait()
                        pltpu.make_async_copy(
                            col_hbm.at[wid, pl.ds(0, SCH)], cidx.at[1 - p],
                            semi).wait()
                @pl.when(s < NSCH3 - 1)
                def _g():
                    gather(1 - p, k - (SCH - 2), bw)
            gather_wait(b)
            scatter(p, k, b)

    scatter_wait((CPW3 - 2) % NB3)
    scatter_wait((CPW3 - 1) % NB3)
    plsc.subcore_barrier()

    for j in range(ZC3):
        r0 = sid * RPT + j * CH3
        pltpu.async_copy(acc_sh.at[pl.ds(r0, CH3)],
                         out_hbm.at[cid, pl.ds(r0, CH3)], semz)
    for j in range(ZC3):
        pltpu.make_async_copy(acc_sh.at[pl.ds(0, CH3)],
                              out_hbm.at[cid, pl.ds(0, CH3)], semz).wait()


def _out_body(p_ref, dis_ref, b_ref, o_ref):
    s = p_ref[0] + p_ref[1]
    o_ref[...] = s * dis_ref[...] + b_ref[...]


_OUT_R = 2000


def _out_call(parts, dis, bias2d):
    return pl.pallas_call(
        _out_body,
        grid=(N // _OUT_R,),
        in_specs=[
            pl.BlockSpec((NC, _OUT_R, D), lambda i: (0, i, 0)),
            pl.BlockSpec((_OUT_R, 1), lambda i: (i, 0)),
            pl.BlockSpec((1, D), lambda i: (0, 0)),
        ],
        out_specs=pl.BlockSpec((_OUT_R, D), lambda i: (i, 0)),
        out_shape=jax.ShapeDtypeStruct((N, D), jnp.float32),
    )(parts, dis, bias2d)


def kernel(x, edge_index, adj_norm_sp, W, bias):
    row = edge_index[0].astype(jnp.int32)
    col = edge_index[1].astype(jnp.int32)
    ppw = EPW - E // NW
    pad = jnp.broadcast_to(N + jnp.arange(ppw, dtype=jnp.int32), (NW, ppw))
    row_p = jnp.concatenate([row.reshape(NW, E // NW), pad], axis=1)
    col_p = jnp.concatenate([col.reshape(NW, E // NW), pad], axis=1)

    deg_part = _deg_kernel(row_p.reshape(NW, CPW, CH))
    degt = jnp.transpose(deg_part)
    hp, dis = _mm_call(x, W, degt)
    parts = _prop_kernel(hp, row_p.reshape(NW, CPW3, CH3),
                         col_p.reshape(NW, CPW3, CH3))
    out = _out_call(parts, dis, bias.reshape(1, D))
    return out


# speedup vs baseline: 1.0749x; 1.0294x over previous
import functools
import jax
import jax.numpy as jnp
from jax import lax
from jax.experimental import pallas as pl
from jax.experimental.pallas import tpu as pltpu
from jax.experimental.pallas import tpu_sc as plsc

N = 10000
E = 320000
D = 128
NC, NS = 2, 16
NW = NC * NS
EPW = 10240
E_PAD = NW * EPW
NPAD = 10240
RPT = NPAD // NS

CH = 128
CPW = EPW // CH
SCH = 8
NSC = CPW // SCH

CH3 = 80
CPW3 = EPW // CH3
NSCH3 = CPW3 // SCH
NB3 = 4
ZC3 = RPT // CH3


def _sc_mesh():
    return plsc.VectorSubcoreMesh(
        core_axis_name="c", subcore_axis_name="s", num_cores=NC, num_subcores=NS
    )


@functools.partial(
    pl.kernel,
    out_type=jax.ShapeDtypeStruct((NC, NPAD), jnp.float32),
    mesh=_sc_mesh(),
    scratch_types=[
        pltpu.VMEM((CPW, CH), jnp.int32),
        pltpu.VMEM((CH,), jnp.float32),
        pltpu.VMEM((RPT,), jnp.float32),
        pltpu.VMEM_SHARED((NPAD,), jnp.float32),
        pltpu.SemaphoreType.DMA,
    ],
)
def _deg_kernel(row_hbm, deg_out, idx_all, ones_v, zer_v, acc_sh, sem):
    cid = lax.axis_index("c")
    sid = lax.axis_index("s")
    wid = sid * NC + cid

    @pl.loop(0, RPT // 16)
    def _zinit(k):
        zer_v[pl.ds(k * 16, 16)] = jnp.zeros((16,), jnp.float32)

    @pl.loop(0, CH // 16)
    def _oinit(k):
        ones_v[pl.ds(k * 16, 16)] = jnp.full((16,), 1.0, jnp.float32)

    pltpu.sync_copy(zer_v, acc_sh.at[pl.ds(sid * RPT, RPT)])
    pltpu.sync_copy(row_hbm.at[wid], idx_all)
    plsc.subcore_barrier()

    @pl.loop(0, NSC)
    def _scat(s):
        for k in range(SCH):
            pltpu.async_copy(ones_v, acc_sh.at[idx_all.at[s * SCH + k]], sem,
                             add=True)
        for k in range(SCH):
            pltpu.make_async_copy(ones_v, acc_sh.at[idx_all.at[s * SCH + k]],
                                  sem).wait()

    plsc.subcore_barrier()
    pltpu.sync_copy(acc_sh.at[pl.ds(sid * RPT, RPT)],
                    deg_out.at[cid, pl.ds(sid * RPT, RPT)])


def _mm_body(x_ref, w_ref, degp_ref, hp_ref, dis_ref):
    h = lax.dot_general(x_ref[...], w_ref[...], (((1,), (1,)), ((), ())),
                        preferred_element_type=jnp.float32)
    r0 = pl.program_id(0) * _MM_R
    deg = (degp_ref[0:1, pl.ds(r0, _MM_R)] +
           degp_ref[1:2, pl.ds(r0, _MM_R)])
    dis_row = jnp.where(deg > 0.0, lax.rsqrt(deg), 0.0)
    dis = jnp.transpose(dis_row, (1, 0))
    hp_ref[...] = h * dis
    dis_ref[...] = dis


_MM_R = 2048


def _mm_call(x, w, degt):
    return pl.pallas_call(
        _mm_body,
        grid=(NPAD // _MM_R,),
        in_specs=[
            pl.BlockSpec((_MM_R, D), lambda i: (i, 0)),
            pl.BlockSpec((D, D), lambda i: (0, 0)),
            pl.BlockSpec((NC, NPAD), lambda i: (0, 0)),
        ],
        out_specs=[
            pl.BlockSpec((_MM_R, D), lambda i: (i, 0)),
            pl.BlockSpec((_MM_R, 1), lambda i: (i, 0)),
        ],
        out_shape=[
            jax.ShapeDtypeStruct((NPAD, D), jnp.float32),
            jax.ShapeDtypeStruct((NPAD, 1), jnp.float32),
        ],
    )(x, w, degt)


@functools.partial(
    pl.kernel,
    out_type=jax.ShapeDtypeStruct((NC, NPAD, D), jnp.float32),
    mesh=_sc_mesh(),
    scratch_types=[
        pltpu.VMEM((2, SCH, CH3), jnp.int32),
        pltpu.VMEM((2, SCH, CH3), jnp.int32),
        pltpu.VMEM((NB3 * CH3, D), jnp.float32),
        pltpu.VMEM_SHARED((NPAD, D), jnp.float32),
        pltpu.SemaphoreType.DMA,
        pltpu.SemaphoreType.DMA,
        pltpu.SemaphoreType.DMA,
        pltpu.SemaphoreType.DMA,
        pltpu.SemaphoreType.DMA,
        pltpu.SemaphoreType.DMA,
        pltpu.SemaphoreType.DMA,
        pltpu.SemaphoreType.DMA,
        pltpu.SemaphoreType.DMA,
        pltpu.SemaphoreType.DMA,
    ],
)
def _prop_kernel(hp_hbm, row_hbm, col_hbm, out_hbm, ridx, cidx, rows_v,
                 acc_sh, semg0, semg1, semg2, semg3, sems0, sems1, sems2,
                 sems3, semi, semz):
    cid = lax.axis_index("c")
    sid = lax.axis_index("s")
    wid = sid * NC + cid
    semg = (semg0, semg1, semg2, semg3)
    sems = (sems0, sems1, sems2, sems3)

    def buf(b):
        return rows_v.at[pl.ds(b * CH3, CH3)]

    def gather(p, slot, b):
        pltpu.async_copy(hp_hbm.at[ridx.at[p, slot]], buf(b), semg[b])

    def gather_wait(b):
        pltpu.make_async_copy(hp_hbm.at[ridx.at[0, 0]], buf(b),
                              semg[b]).wait()

    def scatter(p, slot, b):
        pltpu.async_copy(buf(b), acc_sh.at[cidx.at[p, slot]], sems[b],
                         add=True)

    def scatter_wait(b):
        pltpu.make_async_copy(buf(b), acc_sh.at[cidx.at[0, 0]],
                              sems[b]).wait()

    def load_idx(sc, p):
        pltpu.async_copy(row_hbm.at[wid, pl.ds(sc * SCH, SCH)], ridx.at[p],
                         semi)
        pltpu.async_copy(col_hbm.at[wid, pl.ds(sc * SCH, SCH)], cidx.at[p],
                         semi)

    @pl.loop(0, CH3)
    def _zr(r):
        @pl.loop(0, D // 16)
        def _zc(k):
            rows_v[r, pl.ds(k * 16, 16)] = jnp.zeros((16,), jnp.float32)

    for j in range(ZC3):
        pltpu.async_copy(buf(0), acc_sh.at[pl.ds(sid * RPT + j * CH3, CH3)],
                         semz)
    for j in range(ZC3):
        pltpu.make_async_copy(buf(0), acc_sh.at[pl.ds(0, CH3)], semz).wait()

    pltpu.sync_copy(row_hbm.at[wid, pl.ds(0, SCH)], ridx.at[0])
    pltpu.sync_copy(col_hbm.at[wid, pl.ds(0, SCH)], cidx.at[0])
    load_idx(1, 1)
    gather(0, 0, 0)
    gather(0, 1, 1)
    plsc.subcore_barrier()

    @pl.loop(0, NSCH3)
    def _super(s):
        p = s % 2
        for k in range(SCH):
            b = k % NB3
            bw = (k + 2) % NB3
            if k < 2:
                @pl.when(s > 0)
                def _w():
                    scatter_wait(bw)
            else:
                scatter_wait(bw)
            if k == 2:
                @pl.when((s > 0) & (s < NSCH3 - 1))
                def _pf():
                    load_idx(s + 1, 1 - p)
            if k <= SCH - 3:
                gather(p, k + 2, bw)
            else:
                if k == SCH - 2:
                    @pl.when(s < NSCH3 - 1)
                    def _wi():
                        pltpu.make_async_copy(
                            row_hbm.at[wid, pl.ds(0, SCH)], ridx.at[1 - p],
                            semi).wait()
                        pltpu.make_async_copy(
                            col_hbm.at[wid, pl.ds(0, SCH)], cidx.at[1 - p],
                            semi).wait()
                @pl.when(s < NSCH3 - 1)
                def _g():
                    gather(1 - p, k - (SCH - 2), bw)
            gather_wait(b)
            scatter(p, k, b)

    scatter_wait((CPW3 - 2) % NB3)
    scatter_wait((CPW3 - 1) % NB3)
    plsc.subcore_barrier()

    for j in range(ZC3):
        r0 = sid * RPT + j * CH3
        pltpu.async_copy(acc_sh.at[pl.ds(r0, CH3)],
                         out_hbm.at[cid, pl.ds(r0, CH3)], semz)
    for j in range(ZC3):
        pltpu.make_async_copy(acc_sh.at[pl.ds(0, CH3)],
                              out_hbm.at[cid, pl.ds(0, CH3)], semz).wait()


def _out_body(p_ref, dis_ref, b_ref, o_ref):
    s = p_ref[0] + p_ref[1]
    o_ref[...] = s * dis_ref[...] + b_ref[...]


_OUT_R = 2000


def _out_call(parts, dis, bias2d):
    return pl.pallas_call(
        _out_body,
        grid=(N // _OUT_R,),
        in_specs=[
            pl.BlockSpec((NC, _OUT_R, D), lambda i: (0, i, 0)),
            pl.BlockSpec((_OUT_R, 1), lambda i: (i, 0)),
            pl.BlockSpec((1, D), lambda i: (0, 0)),
        ],
        out_specs=pl.BlockSpec((_OUT_R, D), lambda i: (i, 0)),
        out_shape=jax.ShapeDtypeStruct((N, D), jnp.float32),
    )(parts, dis, bias2d)


def kernel(x, edge_index, adj_norm_sp, W, bias):
    row = edge_index[0].astype(jnp.int32)
    col = edge_index[1].astype(jnp.int32)
    ppw = EPW - E // NW
    pad = jnp.broadcast_to(N + jnp.arange(ppw, dtype=jnp.int32), (NW, ppw))
    row_p = jnp.concatenate([row.reshape(NW, E // NW), pad], axis=1)
    col_p = jnp.concatenate([col.reshape(NW, E // NW), pad], axis=1)

    deg_part = _deg_kernel(row_p.reshape(NW, CPW, CH))
    hp, dis = _mm_call(x, W, deg_part)
    parts = _prop_kernel(hp, row_p.reshape(NW, CPW3, CH3),
                         col_p.reshape(NW, CPW3, CH3))
    out = _out_call(parts, dis, bias.reshape(1, D))
    return out
